# baseline (speedup 1.0000x reference)
import functools

import jax
import jax.numpy as jnp
from jax.experimental import pallas as pl
from jax.experimental.pallas import tpu as pltpu

F = 10
ROW = 128 * F
KBAND = 384
BT = 256


def _band_starts():
    return [min(max(0, 128 * (j - 1)), ROW - KBAND) for j in range(F)]


def _build_band(w):
    s = jnp.asarray(_band_starts(), dtype=jnp.int32)
    q = s[:, None, None] + jnp.arange(KBAND, dtype=jnp.int32)[None, :, None]
    p = (128 * jnp.arange(F, dtype=jnp.int32))[:, None, None] \
        + jnp.arange(128, dtype=jnp.int32)[None, None, :]
    vals = w.astype(jnp.float32)[p % F, q % F]
    return jnp.where(q // F == p // F, vals, 0.0)


def _banded_kernel(x_ref, m_ref, b_ref, o_ref):
    x_blk = x_ref[...]
    starts = _band_starts()
    for j in range(F):
        y = jnp.dot(x_blk[:, starts[j]:starts[j] + KBAND], m_ref[j],
                    preferred_element_type=jnp.float32)
        o_ref[:, 128 * j:128 * (j + 1)] = y + b_ref[:, 128 * j:128 * (j + 1)]


@jax.jit
def _forward(x, w, b):
    n, in_f = x.shape
    flat = n * in_f
    nrows = flat // ROW
    x2 = x.reshape(nrows, ROW)

    m = _build_band(w)
    b_row = jnp.tile(b.astype(jnp.float32), ROW // F).reshape(1, ROW)

    grid = pl.cdiv(nrows, BT)
    y2 = pl.pallas_call(
        _banded_kernel,
        out_shape=jax.ShapeDtypeStruct((nrows, ROW), jnp.float32),
        grid=(grid,),
        in_specs=[
            pl.BlockSpec((BT, ROW), lambda i: (i, 0)),
            pl.BlockSpec((F, KBAND, 128), lambda i: (0, 0, 0)),
            pl.BlockSpec((1, ROW), lambda i: (0, 0)),
        ],
        out_specs=pl.BlockSpec((BT, ROW), lambda i: (i, 0)),
        compiler_params=pltpu.CompilerParams(
            dimension_semantics=("parallel",),
        ),
    )(x2, m, b_row)
    return y2.reshape(16, -1)


def kernel(x, w, b):
    return _forward(x, w, b)


# speedup vs baseline: 13.0490x; 13.0490x over previous
import functools

import numpy as np
import jax
import jax.numpy as jnp
from jax.experimental import pallas as pl
from jax.experimental.pallas import tpu as pltpu

F = 10
CH = 128
ROW = CH * F
NR = 64
BT = NR * CH


def _np_maps():
    t = np.arange(F)[:, None]
    n = np.arange(128)[None, :]
    p = 128 * t + n
    c_of = p // F
    j_of = p % F
    mask = (np.arange(CH)[None, :, None] == c_of[:, None, :])
    oh_out = (j_of[:, :, None] == np.arange(F)[None, None, :])
    oh_bias = (np.arange(ROW)[None, :] % F
               == np.arange(F)[:, None])
    return (mask.astype(np.float32), oh_out.astype(np.float32),
            oh_bias.astype(np.float32))


_MASK, _OH_OUT, _OH_BIAS = _np_maps()


def _scatter_mats(w):
    wv = jnp.einsum("tnj,jk->tnk", _OH_OUT, w.astype(jnp.float32))
    g = jnp.einsum("tcn,tnk->tkcn", _MASK, wv)
    return g.reshape(F, F * CH, 128)


def _relayout_kernel(x_ref, g_ref, b_ref, o_ref):
    x3 = x_ref[...]
    t3 = jnp.transpose(x3, (0, 2, 1))
    s = jnp.concatenate([t3[:, k, :] for k in range(F)], axis=1)
    for t in range(F):
        y = jnp.dot(s, g_ref[t], preferred_element_type=jnp.float32)
        o_ref[:, 128 * t:128 * (t + 1)] = y + b_ref[:, 128 * t:128 * (t + 1)]


@jax.jit
def _forward(x, w, b):
    n, in_f = x.shape
    nrows = n // CH
    x3 = x.reshape(nrows, CH, in_f)

    g = _scatter_mats(w)
    b_row = (b.astype(jnp.float32)[None, :] @ _OH_BIAS)

    y2 = pl.pallas_call(
        _relayout_kernel,
        out_shape=jax.ShapeDtypeStruct((nrows, ROW), jnp.float32),
        grid=(nrows // NR,),
        in_specs=[
            pl.BlockSpec((NR, CH, F), lambda i: (i, 0, 0)),
            pl.BlockSpec((F, F * CH, 128), lambda i: (0, 0, 0)),
            pl.BlockSpec((1, ROW), lambda i: (0, 0)),
        ],
        out_specs=pl.BlockSpec((NR, ROW), lambda i: (i, 0)),
        compiler_params=pltpu.CompilerParams(
            dimension_semantics=("parallel",),
        ),
    )(x3, g, b_row)
    return y2.reshape(16, -1)


def kernel(x, w, b):
    return _forward(x, w, b)
